# baseline (speedup 1.0000x reference)
import functools

import jax
import jax.numpy as jnp
from jax import lax
from jax.experimental import pallas as pl
from jax.experimental.pallas import tpu as pltpu
from jax.experimental.pallas import tpu_sc as plsc

_info = plsc.get_sparse_core_info()
_NC, _NS, _L = _info.num_cores, _info.num_subcores, _info.num_lanes
_NW = _NC * _NS


_NCORES = 1

def _build(B, NF, NE, CD, VOCAB, OFFP, AUXP):
    nw = _NCORES * _NS
    bpw = B // nw
    ng = bpw // _L
    mesh = plsc.VectorSubcoreMesh(core_axis_name="c", subcore_axis_name="s",
                                  num_cores=_NCORES)

    @functools.partial(
        pl.kernel,
        mesh=mesh,
        compiler_params=pltpu.CompilerParams(
            use_tc_tiling_on_sc=False, needs_layout_passes=False),
        out_type=jax.ShapeDtypeStruct((B,), jnp.float32),
        scratch_types=[
            pltpu.VMEM((bpw * NF,), jnp.int32),
            pltpu.VMEM((bpw * CD,), jnp.float32),
            pltpu.VMEM((bpw,), jnp.float32),
            pltpu.VMEM((bpw,), jnp.float32),
            pltpu.VMEM((VOCAB,), jnp.float32),
            pltpu.VMEM((OFFP,), jnp.int32),
            pltpu.VMEM((AUXP,), jnp.float32),
            pltpu.VMEM((bpw,), jnp.float32),
            pltpu.SemaphoreType.DMA,
            pltpu.SemaphoreType.DMA,
            pltpu.SemaphoreType.DMA,
            pltpu.SemaphoreType.DMA,
        ],
    )
    def k(x_h, xc_h, u_h, it_h, tbl_h, off_h, aux_h, out_h,
          xv, xcv, uv, iv, tblv, offv, auxv, outv,
          sem_xc, sem_svd, sem_tbl, sem_x):
        wid = lax.axis_index("s") * _NCORES + lax.axis_index("c")
        base = wid * bpw
        cp_xc = pltpu.async_copy(xc_h.at[pl.ds(base * CD, bpw * CD)],
                                 xcv, sem_xc)
        cp_u = pltpu.async_copy(u_h.at[pl.ds(base, bpw)], uv, sem_svd)
        cp_it = pltpu.async_copy(it_h.at[pl.ds(base, bpw)], iv, sem_svd)
        cp_tbl = pltpu.async_copy(tbl_h, tblv, sem_tbl)
        cp_x = pltpu.async_copy(x_h.at[pl.ds(base * NF, bpw * NF)],
                                xv, sem_x)
        pltpu.sync_copy(off_h, offv)
        pltpu.sync_copy(aux_h, auxv)

        lanes = lax.broadcasted_iota(jnp.int32, (_L,), 0)
        rows = [lanes + g * _L for g in range(ng)]
        bias_s = auxv[pl.ds(CD - CD % _L, _L)][CD % _L] if CD % _L else \
            auxv[pl.ds(CD, _L)][0]
        offvecs = [offv[pl.ds(c * _L, _L)] for c in range(OFFP // _L)]

        cp_x.wait()
        cp_tbl.wait()
        accs = []
        for g in range(ng):
            rnf = rows[g] * NF
            a = jnp.zeros((_L,), jnp.float32) + bias_s
            for f in range(NF):
                xi = plsc.load_gather(xv, [rnf + f])
                xi = xi + offvecs[f // _L][f % _L]
                a = a + plsc.load_gather(tblv, [xi])
            accs.append(a)

        rxc = [rows[g] * CD for g in range(ng)]
        cp_xc.wait()

        def mv_body(t, carry):
            jt = lanes + t
            jt = jnp.where(jt >= CD, jt - CD, jt)
            wv = plsc.load_gather(auxv, [jt])
            return tuple(
                carry[g] + plsc.load_gather(xcv, [rxc[g] + jt]) * wv
                for g in range(ng)
            )

        accs2 = plsc.parallel_loop(0, CD, unroll=8,
                                   carry=tuple(accs))(mv_body)

        cp_u.wait()
        cp_it.wait()
        for g in range(ng):
            sl = pl.ds(g * _L, _L)
            outv[sl] = accs2[g] + uv[sl] + iv[sl]
        pltpu.sync_copy(outv, out_h.at[pl.ds(base, bpw)])

    return k


def kernel(x, svd_emb, x_cont, linear_table, bias, w, offsets):
    B, NF = x.shape
    NE = svd_emb.shape[1] // 2
    CD = x_cont.shape[1]
    VOCAB = linear_table.shape[0]

    tbl = linear_table.reshape(VOCAB)
    OFFP = -(-NF // 16) * 16
    offp = jnp.pad(offsets.astype(jnp.int32), (0, OFFP - NF))
    aux = jnp.concatenate([w, bias])
    AUXP = -(-(CD + 1) // 16) * 16
    aux = jnp.pad(aux, (0, AUXP - (CD + 1)))

    k = _build(B, NF, NE, CD, VOCAB, OFFP, AUXP)
    out = k(x.reshape(-1), x_cont.reshape(-1), svd_emb[:, 0],
            svd_emb[:, NE], tbl, offp, aux)
    return out.reshape(B, 1)


# speedup vs baseline: 1.2321x; 1.2321x over previous
import functools

import jax
import jax.numpy as jnp
from jax import lax
from jax.experimental import pallas as pl
from jax.experimental.pallas import tpu as pltpu
from jax.experimental.pallas import tpu_sc as plsc

_info = plsc.get_sparse_core_info()
_NC, _NS, _L = _info.num_cores, _info.num_subcores, _info.num_lanes
_NW = _NC * _NS


_NCORES = _NC

def _build(B, NF, NE, CD, VOCAB, OFFP, AUXP):
    nw = _NCORES * _NS
    bpw = B // nw
    ng = bpw // _L
    mesh = plsc.VectorSubcoreMesh(core_axis_name="c", subcore_axis_name="s",
                                  num_cores=_NCORES)

    @functools.partial(
        pl.kernel,
        mesh=mesh,
        compiler_params=pltpu.CompilerParams(
            use_tc_tiling_on_sc=False, needs_layout_passes=False),
        out_type=jax.ShapeDtypeStruct((B,), jnp.float32),
        scratch_types=[
            pltpu.VMEM((bpw * NF,), jnp.int32),
            pltpu.VMEM((bpw * CD,), jnp.float32),
            pltpu.VMEM((bpw,), jnp.float32),
            pltpu.VMEM((bpw,), jnp.float32),
            pltpu.VMEM((VOCAB,), jnp.float32),
            pltpu.VMEM((OFFP,), jnp.int32),
            pltpu.VMEM((AUXP,), jnp.float32),
            pltpu.VMEM((bpw,), jnp.float32),
            pltpu.SemaphoreType.DMA,
            pltpu.SemaphoreType.DMA,
            pltpu.SemaphoreType.DMA,
            pltpu.SemaphoreType.DMA,
        ],
    )
    def k(x_h, xc_h, u_h, it_h, tbl_h, off_h, aux_h, out_h,
          xv, xcv, uv, iv, tblv, offv, auxv, outv,
          sem_xc, sem_svd, sem_tbl, sem_x):
        wid = lax.axis_index("s") * _NCORES + lax.axis_index("c")
        base = wid * bpw
        cp_xc = pltpu.async_copy(xc_h.at[pl.ds(base * CD, bpw * CD)],
                                 xcv, sem_xc)
        cp_u = pltpu.async_copy(u_h.at[pl.ds(base, bpw)], uv, sem_svd)
        cp_it = pltpu.async_copy(it_h.at[pl.ds(base, bpw)], iv, sem_svd)
        cp_tbl = pltpu.async_copy(tbl_h, tblv, sem_tbl)
        cp_x = pltpu.async_copy(x_h.at[pl.ds(base * NF, bpw * NF)],
                                xv, sem_x)
        pltpu.sync_copy(off_h, offv)
        pltpu.sync_copy(aux_h, auxv)

        lanes = lax.broadcasted_iota(jnp.int32, (_L,), 0)
        rows = [lanes + g * _L for g in range(ng)]
        bias_s = auxv[pl.ds(CD - CD % _L, _L)][CD % _L] if CD % _L else \
            auxv[pl.ds(CD, _L)][0]
        offvecs = [offv[pl.ds(c * _L, _L)] for c in range(OFFP // _L)]

        cp_x.wait()
        cp_tbl.wait()
        accs = []
        for g in range(ng):
            rnf = rows[g] * NF
            a = jnp.zeros((_L,), jnp.float32) + bias_s
            for f in range(NF):
                xi = plsc.load_gather(xv, [rnf + f])
                xi = xi + offvecs[f // _L][f % _L]
                a = a + plsc.load_gather(tblv, [xi])
            accs.append(a)

        rxc = [rows[g] * CD for g in range(ng)]
        cp_xc.wait()

        def mv_body(t, carry):
            jt = lanes + t
            jt = jnp.where(jt >= CD, jt - CD, jt)
            wv = plsc.load_gather(auxv, [jt])
            return tuple(
                carry[g] + plsc.load_gather(xcv, [rxc[g] + jt]) * wv
                for g in range(ng)
            )

        accs2 = plsc.parallel_loop(0, CD, unroll=8,
                                   carry=tuple(accs))(mv_body)

        cp_u.wait()
        cp_it.wait()
        for g in range(ng):
            sl = pl.ds(g * _L, _L)
            outv[sl] = accs2[g] + uv[sl] + iv[sl]
        pltpu.sync_copy(outv, out_h.at[pl.ds(base, bpw)])

    return k


def kernel(x, svd_emb, x_cont, linear_table, bias, w, offsets):
    B, NF = x.shape
    NE = svd_emb.shape[1] // 2
    CD = x_cont.shape[1]
    VOCAB = linear_table.shape[0]

    tbl = linear_table.reshape(VOCAB)
    OFFP = -(-NF // 16) * 16
    offp = jnp.pad(offsets.astype(jnp.int32), (0, OFFP - NF))
    aux = jnp.concatenate([w, bias])
    AUXP = -(-(CD + 1) // 16) * 16
    aux = jnp.pad(aux, (0, AUXP - (CD + 1)))

    k = _build(B, NF, NE, CD, VOCAB, OFFP, AUXP)
    out = k(x.reshape(-1), x_cont.reshape(-1), svd_emb[:, 0],
            svd_emb[:, NE], tbl, offp, aux)
    return out.reshape(B, 1)


# speedup vs baseline: 1.3677x; 1.1101x over previous
import functools

import jax
import jax.numpy as jnp
from jax import lax
from jax.experimental import pallas as pl
from jax.experimental.pallas import tpu as pltpu
from jax.experimental.pallas import tpu_sc as plsc

_info = plsc.get_sparse_core_info()
_NC, _NS, _L = _info.num_cores, _info.num_subcores, _info.num_lanes
_NW = _NC * _NS


_NCORES = _NC

def _build(B, NF, NE, CD, VOCAB):
    nw = _NCORES * _NS
    bpw = B // nw
    ng = bpw // _L
    mesh = plsc.VectorSubcoreMesh(core_axis_name="c", subcore_axis_name="s",
                                  num_cores=_NCORES)

    @functools.partial(
        pl.kernel,
        mesh=mesh,
        compiler_params=pltpu.CompilerParams(
            use_tc_tiling_on_sc=False, needs_layout_passes=False),
        out_type=jax.ShapeDtypeStruct((B,), jnp.float32),
        scratch_types=[
            pltpu.VMEM((bpw * NF,), jnp.int32),
            pltpu.VMEM((bpw * CD,), jnp.float32),
            pltpu.VMEM((bpw * 2 * NE,), jnp.float32),
            pltpu.VMEM((VOCAB,), jnp.float32),
            pltpu.VMEM((NF,), jnp.int32),
            pltpu.VMEM((CD,), jnp.float32),
            pltpu.VMEM((1,), jnp.float32),
            pltpu.VMEM((bpw,), jnp.float32),
            pltpu.SemaphoreType.DMA,
            pltpu.SemaphoreType.DMA,
            pltpu.SemaphoreType.DMA,
            pltpu.SemaphoreType.DMA,
        ],
    )
    def k(x_h, xc_h, svd_h, tbl_h, off_h, w_h, b_h, out_h,
          xv, xcv, svdv, tblv, offv, auxv, biasv, outv,
          sem_xc, sem_svd, sem_tbl, sem_x):
        wid = lax.axis_index("s") * _NCORES + lax.axis_index("c")
        base = wid * bpw
        cp_xc = pltpu.async_copy(xc_h.at[pl.ds(base * CD, bpw * CD)],
                                 xcv, sem_xc)
        cp_svd = pltpu.async_copy(
            svd_h.at[pl.ds(base * 2 * NE, bpw * 2 * NE)], svdv, sem_svd)
        cp_tbl = pltpu.async_copy(tbl_h, tblv, sem_tbl)
        cp_x = pltpu.async_copy(x_h.at[pl.ds(base * NF, bpw * NF)],
                                xv, sem_x)
        pltpu.sync_copy(off_h, offv)
        pltpu.sync_copy(w_h, auxv)
        pltpu.sync_copy(b_h, biasv)

        lanes = lax.broadcasted_iota(jnp.int32, (_L,), 0)
        zeros16 = jnp.zeros((_L,), jnp.int32)
        rows = [lanes + g * _L for g in range(ng)]
        bias_v = plsc.load_gather(biasv, [zeros16])
        off0 = offv[pl.ds(0, _L)]
        off1 = offv[pl.ds(NF - _L, _L)]
        offsc = [off0[f] if f < _L else off1[f - (NF - _L)]
                 for f in range(NF)]

        cp_x.wait()
        cp_tbl.wait()
        accs = []
        for g in range(ng):
            rnf = rows[g] * NF
            a = bias_v
            for f in range(NF):
                xi = plsc.load_gather(xv, [rnf + f])
                xi = xi + offsc[f]
                a = a + plsc.load_gather(tblv, [xi])
            accs.append(a)

        rxc = [rows[g] * CD for g in range(ng)]
        cp_xc.wait()

        def mv_body(t, carry):
            jt = lanes + t
            jt = jnp.where(jt >= CD, jt - CD, jt)
            wv = plsc.load_gather(auxv, [jt])
            return tuple(
                carry[g] + plsc.load_gather(xcv, [rxc[g] + jt]) * wv
                for g in range(ng)
            )

        accs2 = plsc.parallel_loop(0, CD, unroll=8,
                                   carry=tuple(accs))(mv_body)

        cp_svd.wait()
        for g in range(ng):
            rsv = rows[g] * (2 * NE)
            a = accs2[g] + plsc.load_gather(svdv, [rsv])
            a = a + plsc.load_gather(svdv, [rsv + NE])
            outv[pl.ds(g * _L, _L)] = a
        pltpu.sync_copy(outv, out_h.at[pl.ds(base, bpw)])

    return k


def kernel(x, svd_emb, x_cont, linear_table, bias, w, offsets):
    B, NF = x.shape
    NE = svd_emb.shape[1] // 2
    CD = x_cont.shape[1]
    VOCAB = linear_table.shape[0]

    tbl = linear_table.reshape(VOCAB)
    k = _build(B, NF, NE, CD, VOCAB)
    out = k(x.reshape(-1), x_cont.reshape(-1), svd_emb.reshape(-1),
            tbl, offsets, w, bias)
    return out.reshape(B, 1)
